# baseline (speedup 1.0000x reference)
import jax
import jax.numpy as jnp
from jax import lax
from jax.experimental import pallas as pl
from jax.experimental.pallas import tpu as pltpu


def _fused_body(x_ref, wih_ref, bias_ref, a_ref, whh_ref, wq_ref, bq_ref,
                wd_ref, bd_ref, out_ref, xp_ref, y_ref, abf_ref, whq_ref):
    j = pl.program_id(0)
    nblk = pl.num_programs(0)
    bb, seq, f = x_ref.shape
    bs = a_ref.shape[0]
    hd = whh_ref.shape[0]
    g = 4 * hd

    @pl.when(j == 0)
    def _stage():
        abf_ref[...] = a_ref[...].astype(jnp.bfloat16)
        whq_ref[:, :g] = whh_ref[...].astype(jnp.bfloat16)
        whq_ref[:, g:] = wq_ref[...].astype(jnp.bfloat16)

    x2 = x_ref[...].reshape(bb * seq, f).astype(jnp.bfloat16)
    w = wih_ref[...].astype(jnp.bfloat16)
    z = (jnp.dot(x2, w, preferred_element_type=jnp.float32)
         + bias_ref[...]).astype(jnp.bfloat16)
    xp_ref[:, pl.ds(j * bb, bb), :] = jnp.transpose(
        z.reshape(bb, seq, g), (1, 0, 2))

    @pl.when(j == nblk - 1)
    def _recur():
        bq = bq_ref[...]
        wd = wd_ref[...]
        h = None
        c = None
        for t in range(seq):
            if t == 0:
                z4 = xp_ref[0].astype(jnp.float32)
                q_t = jnp.broadcast_to(jnp.tanh(bq), (bs, hd))
            else:
                zr = jnp.dot(h.astype(jnp.bfloat16), whq_ref[...],
                             preferred_element_type=jnp.float32)
                z4 = zr[:, :g] + xp_ref[t].astype(jnp.float32)
                q_t = jnp.tanh(zr[:, g:] + bq)
            if_t = 0.5 * jnp.tanh(0.5 * z4[:, :2 * hd]) + 0.5
            i_t = if_t[:, :hd]
            f_t = if_t[:, hd:]
            g_t = jnp.tanh(z4[:, 2 * hd:3 * hd])
            o_t = 0.5 * jnp.tanh(0.5 * z4[:, 3 * hd:]) + 0.5
            aq = jnp.dot(abf_ref[...], q_t.astype(jnp.bfloat16),
                         preferred_element_type=jnp.float32)
            ca = c + aq if t > 0 else aq
            c = f_t * ca + i_t * g_t
            h = o_t * jnp.tanh(c)
            y_ref[t:t + 1, :] = lax.dot_general(
                wd, h, (((1,), (1,)), ((), ())),
                preferred_element_type=jnp.float32)
        eye = (lax.broadcasted_iota(jnp.int32, (seq, seq), 0) ==
               lax.broadcasted_iota(jnp.int32, (seq, seq), 1)
               ).astype(jnp.float32)
        out_ref[...] = lax.dot_general(
            y_ref[...], eye, (((0,), (0,)), ((), ())),
            preferred_element_type=jnp.float32) + bd_ref[...]


def kernel(x, A, w_ih, w_hh, bias, w_q, b_q, w_d, b_d):
    bs, seq, f = x.shape
    hd = w_hh.shape[0]
    g = 4 * hd

    bias2 = bias.reshape(1, g)
    bq2 = b_q.reshape(1, hd)
    wd2 = w_d.reshape(1, hd)
    bd2 = b_d.reshape(1, 1)

    bb = min(32, bs)
    nblk = bs // bb
    out = pl.pallas_call(
        _fused_body,
        out_shape=jax.ShapeDtypeStruct((bs, seq), jnp.float32),
        grid=(nblk,),
        in_specs=[
            pl.BlockSpec((bb, seq, f), lambda j: (j, 0, 0)),
            pl.BlockSpec((f, g), lambda j: (0, 0)),
            pl.BlockSpec((1, g), lambda j: (0, 0)),
            pl.BlockSpec((bs, bs), lambda j: (0, 0)),
            pl.BlockSpec((hd, g), lambda j: (0, 0)),
            pl.BlockSpec((hd, hd), lambda j: (0, 0)),
            pl.BlockSpec((1, hd), lambda j: (0, 0)),
            pl.BlockSpec((1, hd), lambda j: (0, 0)),
            pl.BlockSpec((1, 1), lambda j: (0, 0)),
        ],
        out_specs=pl.BlockSpec((bs, seq), lambda j: (0, 0)),
        scratch_shapes=[pltpu.VMEM((seq, bs, g), jnp.bfloat16),
                        pltpu.VMEM((seq, bs), jnp.float32),
                        pltpu.VMEM((bs, bs), jnp.bfloat16),
                        pltpu.VMEM((hd, 5 * hd), jnp.bfloat16)],
        compiler_params=pltpu.CompilerParams(
            dimension_semantics=("arbitrary",)),
    )(x, w_ih, bias2, A, w_hh, w_q, bq2, wd2, bd2)

    return out[:, :, None]


# speedup vs baseline: 1.0357x; 1.0357x over previous
import jax
import jax.numpy as jnp
from jax import lax
from jax.experimental import pallas as pl
from jax.experimental.pallas import tpu as pltpu


def _fused_body(x_ref, wih_ref, bias_ref, a_ref, whh_ref, wq_ref, bq_ref,
                wd_ref, bd_ref, out_ref, xp_ref, y_ref, abf_ref, whq_ref):
    j = pl.program_id(0)
    nblk = pl.num_programs(0)
    bb, seq, f = x_ref.shape
    bs = a_ref.shape[0]
    hd = whh_ref.shape[0]
    g = 4 * hd

    @pl.when(j == 0)
    def _stage():
        abf_ref[...] = a_ref[...].astype(jnp.bfloat16)
        whq_ref[:, :g] = whh_ref[...].astype(jnp.bfloat16)
        whq_ref[:, g:] = wq_ref[...].astype(jnp.bfloat16)

    x2 = x_ref[...].reshape(bb * seq, f).astype(jnp.bfloat16)
    w = wih_ref[...].astype(jnp.bfloat16)
    z = (jnp.dot(x2, w, preferred_element_type=jnp.float32)
         + bias_ref[...]).astype(jnp.bfloat16)
    xp_ref[:, pl.ds(j * bb, bb), :] = jnp.transpose(
        z.reshape(bb, seq, g), (1, 0, 2))

    @pl.when(j == nblk - 1)
    def _recur():
        bq = bq_ref[...]
        wd = wd_ref[...]
        h = None
        c = None
        for t in range(seq):
            if t == 0:
                z4 = xp_ref[0].astype(jnp.float32)
                q_t = jnp.broadcast_to(jnp.tanh(bq), (bs, hd))
            else:
                zr = jnp.dot(h.astype(jnp.bfloat16), whq_ref[...],
                             preferred_element_type=jnp.float32)
                z4 = zr[:, :g] + xp_ref[t].astype(jnp.float32)
                q_t = jnp.tanh(zr[:, g:] + bq)
            if_t = 0.5 * jnp.tanh(0.5 * z4[:, :2 * hd]) + 0.5
            i_t = if_t[:, :hd]
            f_t = if_t[:, hd:]
            g_t = jnp.tanh(z4[:, 2 * hd:3 * hd])
            o_t = 0.5 * jnp.tanh(0.5 * z4[:, 3 * hd:]) + 0.5
            aq = jnp.dot(abf_ref[...], q_t.astype(jnp.bfloat16),
                         preferred_element_type=jnp.float32)
            ca = c + aq if t > 0 else aq
            c = f_t * ca + i_t * g_t
            h = o_t * jnp.tanh(c)
            y_ref[t:t + 1, :] = lax.dot_general(
                wd, h, (((1,), (1,)), ((), ())),
                preferred_element_type=jnp.float32)
        eye = (lax.broadcasted_iota(jnp.int32, (seq, seq), 0) ==
               lax.broadcasted_iota(jnp.int32, (seq, seq), 1)
               ).astype(jnp.float32)
        out_ref[...] = lax.dot_general(
            y_ref[...], eye, (((0,), (0,)), ((), ())),
            preferred_element_type=jnp.float32) + bd_ref[...]


def kernel(x, A, w_ih, w_hh, bias, w_q, b_q, w_d, b_d):
    bs, seq, f = x.shape
    hd = w_hh.shape[0]
    g = 4 * hd

    bias2 = bias.reshape(1, g)
    bq2 = b_q.reshape(1, hd)
    wd2 = w_d.reshape(1, hd)
    bd2 = b_d.reshape(1, 1)

    bb = min(128, bs)
    nblk = bs // bb
    out = pl.pallas_call(
        _fused_body,
        out_shape=jax.ShapeDtypeStruct((bs, seq), jnp.float32),
        grid=(nblk,),
        in_specs=[
            pl.BlockSpec((bb, seq, f), lambda j: (j, 0, 0)),
            pl.BlockSpec((f, g), lambda j: (0, 0)),
            pl.BlockSpec((1, g), lambda j: (0, 0)),
            pl.BlockSpec((bs, bs), lambda j: (0, 0)),
            pl.BlockSpec((hd, g), lambda j: (0, 0)),
            pl.BlockSpec((hd, hd), lambda j: (0, 0)),
            pl.BlockSpec((1, hd), lambda j: (0, 0)),
            pl.BlockSpec((1, hd), lambda j: (0, 0)),
            pl.BlockSpec((1, 1), lambda j: (0, 0)),
        ],
        out_specs=pl.BlockSpec((bs, seq), lambda j: (0, 0)),
        scratch_shapes=[pltpu.VMEM((seq, bs, g), jnp.bfloat16),
                        pltpu.VMEM((seq, bs), jnp.float32),
                        pltpu.VMEM((bs, bs), jnp.bfloat16),
                        pltpu.VMEM((hd, 5 * hd), jnp.bfloat16)],
        compiler_params=pltpu.CompilerParams(
            dimension_semantics=("arbitrary",)),
    )(x, w_ih, bias2, A, w_hh, w_q, bq2, wd2, bd2)

    return out[:, :, None]


# speedup vs baseline: 1.0825x; 1.0452x over previous
import jax
import jax.numpy as jnp
from jax import lax
from jax.experimental import pallas as pl
from jax.experimental.pallas import tpu as pltpu


def _fused_body(x_ref, wih_ref, bias_ref, a_ref, whh_ref, wq_ref, bq_ref,
                wd_ref, bd_ref, out_ref, xp_ref, y_ref, abf_ref, whq_ref):
    j = pl.program_id(0)
    nblk = pl.num_programs(0)
    bb, seq, f = x_ref.shape
    bs = a_ref.shape[0]
    hd = whh_ref.shape[0]
    g = 4 * hd

    @pl.when(j == 0)
    def _stage():
        abf_ref[...] = a_ref[...].astype(jnp.bfloat16)
        whq_ref[:, :g] = whh_ref[...].astype(jnp.bfloat16)
        whq_ref[:, g:] = wq_ref[...].astype(jnp.bfloat16)

    x2 = x_ref[...].reshape(bb * seq, f).astype(jnp.bfloat16)
    w = wih_ref[...].astype(jnp.bfloat16)
    z = (jnp.dot(x2, w, preferred_element_type=jnp.float32)
         + bias_ref[...]).astype(jnp.bfloat16)
    xp_ref[:, pl.ds(j * bb, bb), :] = jnp.transpose(
        z.reshape(bb, seq, g), (1, 0, 2))

    @pl.when(j == nblk - 1)
    def _recur():
        bq = bq_ref[...]
        wd = wd_ref[...]
        h = None
        c = None
        for t in range(seq):
            if t == 0:
                z4 = xp_ref[0].astype(jnp.float32)
                q_t = jnp.broadcast_to(jnp.tanh(bq), (bs, hd))
            else:
                zr = jnp.dot(h.astype(jnp.bfloat16), whq_ref[...],
                             preferred_element_type=jnp.float32)
                z4 = zr[:, :g] + xp_ref[t].astype(jnp.float32)
                q_t = jnp.tanh(zr[:, g:] + bq)
            if_t = 0.5 * jnp.tanh(0.5 * z4[:, :2 * hd]) + 0.5
            i_t = if_t[:, :hd]
            f_t = if_t[:, hd:]
            g_t = jnp.tanh(z4[:, 2 * hd:3 * hd])
            o_t = 0.5 * jnp.tanh(0.5 * z4[:, 3 * hd:]) + 0.5
            aq = jnp.dot(abf_ref[...], q_t.astype(jnp.bfloat16),
                         preferred_element_type=jnp.float32)
            ca = c + aq if t > 0 else aq
            c = f_t * ca + i_t * g_t
            h = o_t * jnp.tanh(c)
            y_ref[t:t + 1, :] = lax.dot_general(
                wd, h, (((1,), (1,)), ((), ())),
                preferred_element_type=jnp.float32)
        eye = (lax.broadcasted_iota(jnp.int32, (seq, seq), 0) ==
               lax.broadcasted_iota(jnp.int32, (seq, seq), 1)
               ).astype(jnp.float32)
        out_ref[...] = lax.dot_general(
            y_ref[...], eye, (((0,), (0,)), ((), ())),
            preferred_element_type=jnp.float32) + bd_ref[...]


def kernel(x, A, w_ih, w_hh, bias, w_q, b_q, w_d, b_d):
    bs, seq, f = x.shape
    hd = w_hh.shape[0]
    g = 4 * hd

    bias2 = bias.reshape(1, g)
    bq2 = b_q.reshape(1, hd)
    wd2 = w_d.reshape(1, hd)
    bd2 = b_d.reshape(1, 1)

    bb = min(64, bs)
    nblk = bs // bb
    out = pl.pallas_call(
        _fused_body,
        out_shape=jax.ShapeDtypeStruct((bs, seq), jnp.float32),
        grid=(nblk,),
        in_specs=[
            pl.BlockSpec((bb, seq, f), lambda j: (j, 0, 0)),
            pl.BlockSpec((f, g), lambda j: (0, 0)),
            pl.BlockSpec((1, g), lambda j: (0, 0)),
            pl.BlockSpec((bs, bs), lambda j: (0, 0)),
            pl.BlockSpec((hd, g), lambda j: (0, 0)),
            pl.BlockSpec((hd, hd), lambda j: (0, 0)),
            pl.BlockSpec((1, hd), lambda j: (0, 0)),
            pl.BlockSpec((1, hd), lambda j: (0, 0)),
            pl.BlockSpec((1, 1), lambda j: (0, 0)),
        ],
        out_specs=pl.BlockSpec((bs, seq), lambda j: (0, 0)),
        scratch_shapes=[pltpu.VMEM((seq, bs, g), jnp.bfloat16),
                        pltpu.VMEM((seq, bs), jnp.float32),
                        pltpu.VMEM((bs, bs), jnp.bfloat16),
                        pltpu.VMEM((hd, 5 * hd), jnp.bfloat16)],
        compiler_params=pltpu.CompilerParams(
            dimension_semantics=("arbitrary",)),
    )(x, w_ih, bias2, A, w_hh, w_q, bq2, wd2, bd2)

    return out[:, :, None]


# speedup vs baseline: 1.1193x; 1.0340x over previous
import jax
import jax.numpy as jnp
from jax import lax
from jax.experimental import pallas as pl
from jax.experimental.pallas import tpu as pltpu


def _fused_body(x_ref, wih_ref, bias_ref, a_ref, whh_ref, wq_ref, bq_ref,
                wd_ref, bd_ref, out_ref, xp_ref, y_ref, abf_ref, whq_ref,
                hs_ref):
    j = pl.program_id(0)
    nblk = pl.num_programs(0)
    bb, seq, f = x_ref.shape
    bs = a_ref.shape[0]
    hd = whh_ref.shape[0]
    g = 4 * hd

    @pl.when(j == 0)
    def _stage():
        abf_ref[...] = a_ref[...].astype(jnp.bfloat16)
        whq_ref[:, :g] = whh_ref[...].astype(jnp.bfloat16)
        whq_ref[:, g:] = wq_ref[...].astype(jnp.bfloat16)

    x2 = x_ref[...].reshape(bb * seq, f).astype(jnp.bfloat16)
    w = wih_ref[...].astype(jnp.bfloat16)
    z = (jnp.dot(x2, w, preferred_element_type=jnp.float32)
         + bias_ref[...]).astype(jnp.bfloat16)
    xp_ref[:, pl.ds(j * bb, bb), :] = jnp.transpose(
        z.reshape(bb, seq, g), (1, 0, 2))

    @pl.when(j == nblk - 1)
    def _recur():
        bq = bq_ref[...]
        wd = wd_ref[...]
        h = None
        c = None
        for t in range(seq):
            if t == 0:
                z4 = xp_ref[0].astype(jnp.float32)
                q_t = jnp.broadcast_to(jnp.tanh(bq), (bs, hd))
            else:
                zr = jnp.dot(h.astype(jnp.bfloat16), whq_ref[...],
                             preferred_element_type=jnp.float32)
                z4 = zr[:, :g] + xp_ref[t].astype(jnp.float32)
                q_t = jnp.tanh(zr[:, g:] + bq)
            if_t = 0.5 * jnp.tanh(0.5 * z4[:, :2 * hd]) + 0.5
            i_t = if_t[:, :hd]
            f_t = if_t[:, hd:]
            g_t = jnp.tanh(z4[:, 2 * hd:3 * hd])
            o_t = 0.5 * jnp.tanh(0.5 * z4[:, 3 * hd:]) + 0.5
            aq = jnp.dot(abf_ref[...], q_t.astype(jnp.bfloat16),
                         preferred_element_type=jnp.float32)
            ca = c + aq if t > 0 else aq
            c = f_t * ca + i_t * g_t
            h = o_t * jnp.tanh(c)
            hs_ref[t] = h.astype(jnp.bfloat16)
        wdb = wd.astype(jnp.bfloat16)
        for t in range(seq):
            y_ref[t:t + 1, :] = lax.dot_general(
                wdb, hs_ref[t], (((1,), (1,)), ((), ())),
                preferred_element_type=jnp.float32)
        eye = (lax.broadcasted_iota(jnp.int32, (seq, seq), 0) ==
               lax.broadcasted_iota(jnp.int32, (seq, seq), 1)
               ).astype(jnp.float32)
        out_ref[...] = lax.dot_general(
            y_ref[...], eye, (((0,), (0,)), ((), ())),
            preferred_element_type=jnp.float32) + bd_ref[...]


def kernel(x, A, w_ih, w_hh, bias, w_q, b_q, w_d, b_d):
    bs, seq, f = x.shape
    hd = w_hh.shape[0]
    g = 4 * hd

    bias2 = bias.reshape(1, g)
    bq2 = b_q.reshape(1, hd)
    wd2 = w_d.reshape(1, hd)
    bd2 = b_d.reshape(1, 1)

    bb = min(64, bs)
    nblk = bs // bb
    out = pl.pallas_call(
        _fused_body,
        out_shape=jax.ShapeDtypeStruct((bs, seq), jnp.float32),
        grid=(nblk,),
        in_specs=[
            pl.BlockSpec((bb, seq, f), lambda j: (j, 0, 0)),
            pl.BlockSpec((f, g), lambda j: (0, 0)),
            pl.BlockSpec((1, g), lambda j: (0, 0)),
            pl.BlockSpec((bs, bs), lambda j: (0, 0)),
            pl.BlockSpec((hd, g), lambda j: (0, 0)),
            pl.BlockSpec((hd, hd), lambda j: (0, 0)),
            pl.BlockSpec((1, hd), lambda j: (0, 0)),
            pl.BlockSpec((1, hd), lambda j: (0, 0)),
            pl.BlockSpec((1, 1), lambda j: (0, 0)),
        ],
        out_specs=pl.BlockSpec((bs, seq), lambda j: (0, 0)),
        scratch_shapes=[pltpu.VMEM((seq, bs, g), jnp.bfloat16),
                        pltpu.VMEM((seq, bs), jnp.float32),
                        pltpu.VMEM((bs, bs), jnp.bfloat16),
                        pltpu.VMEM((hd, 5 * hd), jnp.bfloat16),
                        pltpu.VMEM((seq, bs, hd), jnp.bfloat16)],
        compiler_params=pltpu.CompilerParams(
            dimension_semantics=("arbitrary",)),
    )(x, w_ih, bias2, A, w_hh, w_q, bq2, wd2, bd2)

    return out[:, :, None]
